# Initial kernel scaffold; baseline (speedup 1.0000x reference)
import jax
import jax.numpy as jnp
from jax.experimental import pallas as pl


def kernel(ids, bigram_weight, tri_weight):
    raise NotImplementedError("write your pallas kernel here")


# speedup vs baseline: 1.4792x; 1.4792x over previous
import functools

import jax
import jax.numpy as jnp
from jax.experimental import pallas as pl
from jax.experimental.pallas import tpu as pltpu
from jax.experimental.pallas import tpu_sc as plsc

_VOCAB = 1000000
_DIM = 32
_L = 16
_W = 128


def _hash_body(ids_ref, bi_ref, tri_ref):
    x = ids_ref[...]
    z1 = jnp.zeros((x.shape[0], 1), jnp.int32)
    prev = jnp.concatenate([z1, x[:, :-1]], axis=1)
    z2 = jnp.zeros((x.shape[0], 2), jnp.int32)
    prev2 = jnp.concatenate([z2, x[:, :-2]], axis=1)
    s = prev * 131 + x
    bi_ref[...] = s % _VOCAB
    tri_ref[...] = (prev2 * 173 + s) % _VOCAB


def _hash_indices(ids):
    n, m = ids.shape
    blk = 512
    grid = (n // blk,)
    spec = pl.BlockSpec((blk, m), lambda i: (i, 0))
    return pl.pallas_call(
        _hash_body,
        grid=grid,
        in_specs=[spec],
        out_specs=[spec, spec],
        out_shape=[
            jax.ShapeDtypeStruct((n, m), jnp.int32),
            jax.ShapeDtypeStruct((n, m), jnp.int32),
        ],
    )(ids)


def _sc_gather_add(bi_idx, tri_idx, bw, tw, total):
    mesh = plsc.VectorSubcoreMesh(core_axis_name="c", subcore_axis_name="s")

    @functools.partial(
        pl.kernel,
        out_type=jax.ShapeDtypeStruct((total, _DIM), jnp.float32),
        mesh=mesh,
        compiler_params=pltpu.CompilerParams(use_tc_tiling_on_sc=False),
        scratch_types=[
            pltpu.VMEM((_W, _DIM), jnp.float32),
            pltpu.VMEM((_W, _DIM), jnp.float32),
            pltpu.SemaphoreType.DMA,
            pltpu.SemaphoreType.DMA,
        ],
    )
    def k(bi_hbm, tri_hbm, bw_hbm, tw_hbm, out_hbm, rows_bi, rows_tri, s1, s2):
        def body(bi_v, tri_v, out_v):
            c1 = pltpu.async_copy(bw_hbm.at[bi_v.at[0]], rows_bi, s1)
            c2 = pltpu.async_copy(tw_hbm.at[tri_v.at[0]], rows_tri, s2)
            c1.wait()
            c2.wait()

            @pl.loop(0, _W)
            def _(r):
                out_v[r, pl.ds(0, _L)] = (
                    rows_bi[r, pl.ds(0, _L)] + rows_tri[r, pl.ds(0, _L)]
                )
                out_v[r, pl.ds(_L, _L)] = (
                    rows_bi[r, pl.ds(_L, _L)] + rows_tri[r, pl.ds(_L, _L)]
                )

        pltpu.emit_pipeline(
            body,
            grid=(total // _W,),
            in_specs=[
                pl.BlockSpec((1, _W), lambda i: (0, i)),
                pl.BlockSpec((1, _W), lambda i: (0, i)),
            ],
            out_specs=[pl.BlockSpec((_W, _DIM), lambda i: (i, 0))],
            core_axis_name=("c", "s"),
            dimension_semantics=(pltpu.PARALLEL,),
        )(bi_hbm, tri_hbm, out_hbm)

    return k(bi_idx, tri_idx, bw, tw)


def kernel(ids, bigram_weight, tri_weight):
    ids = ids.astype(jnp.int32)
    n, m = ids.shape
    total = n * m
    bi_idx, tri_idx = _hash_indices(ids)
    out = _sc_gather_add(
        bi_idx.reshape(1, total),
        tri_idx.reshape(1, total),
        bigram_weight,
        tri_weight,
        total,
    )
    return out.reshape(n, m, _DIM)
